# speedup vs baseline: 10.2021x; 10.2021x over previous
import jax
import jax.numpy as jnp
from jax import lax
from jax.experimental import pallas as pl
from jax.experimental.pallas import tpu as pltpu

_N = 1000
_E = 100
_NP = 1024
_EP = 128


def _fused_body(edge_ref, x_ref,
                wl0, bl0, wr0, wl1, bl1, wr1, wl2, bl2, wr2,
                wl3, bl3, wr3, wl4, bl4, wr4, out_ref):
    src = edge_ref[0:1, :]
    dst = edge_ref[1:2, :]
    row_ids = lax.broadcasted_iota(jnp.int32, (_NP, 1), 0)

    p_src_t = (row_ids == src).astype(jnp.float32)
    p_dst = (row_ids == dst).astype(jnp.float32)
    count = jnp.sum(p_dst, axis=1, keepdims=True)
    p_dst = p_dst / jnp.maximum(count, 1.0)

    weights = ((wl0, bl0, wr0), (wl1, bl1, wr1), (wl2, bl2, wr2),
               (wl3, bl3, wr3), (wl4, bl4, wr4))

    h = x_ref[...]
    for i, (wl, bl, wr) in enumerate(weights):
        if i > 0:
            h = jnp.maximum(h, 0.0)
        xj = lax.dot_general(p_src_t, h, (((0,), (0,)), ((), ())),
                             preferred_element_type=jnp.float32)
        m = lax.dot_general(xj, wl[...], (((1,), (1,)), ((), ())),
                            preferred_element_type=jnp.float32)
        aggl = lax.dot_general(p_dst, m, (((1,), (0,)), ((), ())),
                               preferred_element_type=jnp.float32)
        dense = lax.dot_general(h, wr[...], (((1,), (1,)), ((), ())),
                                preferred_element_type=jnp.float32)
        h = aggl + dense + bl[...]
    out_ref[...] = h


def kernel(L_x_, L_edge_index_, L_self_modules_convs_modules_0_modules_lin_l_parameters_weight_, L_self_modules_convs_modules_0_modules_lin_l_parameters_bias_, L_self_modules_convs_modules_0_modules_lin_r_parameters_weight_, L_self_modules_convs_modules_1_modules_lin_l_parameters_weight_, L_self_modules_convs_modules_1_modules_lin_l_parameters_bias_, L_self_modules_convs_modules_1_modules_lin_r_parameters_weight_, L_self_modules_convs_modules_2_modules_lin_l_parameters_weight_, L_self_modules_convs_modules_2_modules_lin_l_parameters_bias_, L_self_modules_convs_modules_2_modules_lin_r_parameters_weight_, L_self_modules_convs_modules_3_modules_lin_l_parameters_weight_, L_self_modules_convs_modules_3_modules_lin_l_parameters_bias_, L_self_modules_convs_modules_3_modules_lin_r_parameters_weight_, L_self_modules_convs_modules_4_modules_lin_l_parameters_weight_, L_self_modules_convs_modules_4_modules_lin_l_parameters_bias_, L_self_modules_convs_modules_4_modules_lin_r_parameters_weight_):
    x = L_x_
    edge = L_edge_index_
    edge_p = jnp.full((8, _EP), -1, dtype=jnp.int32)
    edge_p = edge_p.at[:2, :_E].set(edge)
    x_p = jnp.zeros((_NP, 128), dtype=jnp.float32).at[:_N, :].set(x)

    ws = [
        L_self_modules_convs_modules_0_modules_lin_l_parameters_weight_,
        L_self_modules_convs_modules_0_modules_lin_l_parameters_bias_.reshape(1, -1),
        L_self_modules_convs_modules_0_modules_lin_r_parameters_weight_,
        L_self_modules_convs_modules_1_modules_lin_l_parameters_weight_,
        L_self_modules_convs_modules_1_modules_lin_l_parameters_bias_.reshape(1, -1),
        L_self_modules_convs_modules_1_modules_lin_r_parameters_weight_,
        L_self_modules_convs_modules_2_modules_lin_l_parameters_weight_,
        L_self_modules_convs_modules_2_modules_lin_l_parameters_bias_.reshape(1, -1),
        L_self_modules_convs_modules_2_modules_lin_r_parameters_weight_,
        L_self_modules_convs_modules_3_modules_lin_l_parameters_weight_,
        L_self_modules_convs_modules_3_modules_lin_l_parameters_bias_.reshape(1, -1),
        L_self_modules_convs_modules_3_modules_lin_r_parameters_weight_,
        L_self_modules_convs_modules_4_modules_lin_l_parameters_weight_,
        L_self_modules_convs_modules_4_modules_lin_l_parameters_bias_.reshape(1, -1),
        L_self_modules_convs_modules_4_modules_lin_r_parameters_weight_,
    ]

    out = pl.pallas_call(
        _fused_body,
        out_shape=jax.ShapeDtypeStruct((_NP, 256), jnp.float32),
    )(edge_p, x_p, *ws)
    return out[:_N]
